# baseline (speedup 1.0000x reference)
import functools

import jax
import jax.numpy as jnp
from jax import lax
from jax.experimental import pallas as pl
from jax.experimental.pallas import tpu as pltpu
from jax.experimental.pallas import tpu_sc as plsc

_B = 4096
_S = 200
_E = 64
_N = _B * _S
_NW = 32
_PER_W = _N // _NW
_C = _S
_NCHUNK = _PER_W // _C


def _emb_body(x_hbm, tok_hbm, pos_hbm, out_hbm, pos_v, idx_v, rows_v, sem):
    wid = lax.axis_index("s") * 2 + lax.axis_index("c")
    base = wid * _PER_W
    pltpu.sync_copy(pos_hbm, pos_v)

    def chunk(g, carry):
        off = base + g * _C
        pltpu.sync_copy(x_hbm.at[pl.ds(off, _C)], idx_v)
        pltpu.async_copy(tok_hbm.at[idx_v], rows_v, sem).wait()

        def radd(r, c2):
            for c in range(_E // 16):
                sl = pl.ds(c * 16, 16)
                rows_v[r, sl] = rows_v[r, sl] + pos_v[r, sl]
            return c2

        lax.fori_loop(0, _C, radd, 0)
        pltpu.sync_copy(rows_v, out_hbm.at[pl.ds(off, _C)])
        return carry

    lax.fori_loop(0, _NCHUNK, chunk, 0)


@jax.jit
def _emb(xf, token_emb, pos_emb):
    mesh = plsc.VectorSubcoreMesh(core_axis_name="c", subcore_axis_name="s")
    f = functools.partial(
        pl.kernel,
        mesh=mesh,
        compiler_params=pltpu.CompilerParams(use_tc_tiling_on_sc=False),
        out_type=jax.ShapeDtypeStruct((_N, _E), jnp.float32),
        scratch_types=[
            pltpu.VMEM((_S, _E), jnp.float32),
            pltpu.VMEM((_C,), jnp.int32),
            pltpu.VMEM((_C, _E), jnp.float32),
            pltpu.SemaphoreType.DMA,
        ],
    )(_emb_body)
    return f(xf, token_emb, pos_emb)


def kernel(x, token_emb, pos_emb):
    xf = x.reshape(-1)
    out = _emb(xf, token_emb, pos_emb)
    return out.reshape(x.shape[0], x.shape[1], _E)


# speedup vs baseline: 1.0306x; 1.0306x over previous
import functools

import jax
import jax.numpy as jnp
from jax import lax
from jax.experimental import pallas as pl
from jax.experimental.pallas import tpu as pltpu
from jax.experimental.pallas import tpu_sc as plsc

_B = 4096
_S = 200
_E = 64
_N = _B * _S
_NW = 32
_PER_W = _N // _NW
_C = 128
_NCHUNK = _PER_W // _C


def _emb_body(x_hbm, tok2_hbm, pos2_hbm, out_hbm,
              pos_v, idx0, idx1, pidx0, pidx1,
              rows0, rows1, outb0, outb1,
              gsem0, gsem1, osem0, osem1):
    idx = (idx0, idx1)
    pidx = (pidx0, pidx1)
    rows = (rows0, rows1)
    outb = (outb0, outb1)
    gsem = (gsem0, gsem1)
    osem = (osem0, osem1)

    wid = lax.axis_index("s") * 2 + lax.axis_index("c")
    base = wid * _PER_W
    pltpu.sync_copy(pos2_hbm, pos_v)

    def launch(g, b):
        pltpu.sync_copy(x_hbm.at[pl.ds(base + g * _C, _C)], idx[b])
        for v in range(_C // 16):
            sl = pl.ds(v * 16, 16)
            pidx[b][sl] = jax.lax.shift_right_logical(idx[b][sl], 1)
        pltpu.async_copy(tok2_hbm.at[pidx[b]], rows[b], gsem[b])

    launch(0, 0)

    def pair(g2, carry):
        for b in range(2):
            g = g2 * 2 + b
            nb = 1 - b
            nxt = g + 1

            @pl.when(nxt < _NCHUNK)
            def _prefetch():
                launch(nxt, nb)

            pltpu.make_async_copy(tok2_hbm.at[pidx[b]], rows[b],
                                  gsem[b]).wait()

            @pl.when(g >= 2)
            def _drain_store():
                pltpu.make_async_copy(
                    outb[b], out_hbm.at[pl.ds(base, _C)], osem[b]
                ).wait()

            phase = lax.rem(g * _C, _S)

            def compact(g16, c2):
                r0 = g16 * 16
                hv = (idx[b][pl.ds(r0, 16)] & 1) * _E
                for ln in range(16):
                    r = r0 + ln
                    half = hv[ln]
                    pr = phase + r
                    for c in range(_E // 16):
                        dst = pl.ds(c * 16, 16)
                        src = pl.dslice(half + c * 16, 16)
                        outb[b][r, dst] = rows[b][r, src] + pos_v[pr, dst]
                return c2

            lax.fori_loop(0, _C // 16, compact, 0)
            pltpu.async_copy(outb[b], out_hbm.at[pl.ds(base + g * _C, _C)],
                             osem[b])
        return carry

    lax.fori_loop(0, _NCHUNK // 2, pair, 0)

    pltpu.make_async_copy(outb0, out_hbm.at[pl.ds(base, _C)], osem0).wait()
    pltpu.make_async_copy(outb1, out_hbm.at[pl.ds(base, _C)], osem1).wait()


@jax.jit
def _emb(xf, tok2, pos2):
    mesh = plsc.VectorSubcoreMesh(core_axis_name="c", subcore_axis_name="s")
    f = functools.partial(
        pl.kernel,
        mesh=mesh,
        out_type=jax.ShapeDtypeStruct((_N, _E), jnp.float32),
        scratch_types=[
            pltpu.VMEM((2 * _S, _E), jnp.float32),
            pltpu.VMEM((_C,), jnp.int32),
            pltpu.VMEM((_C,), jnp.int32),
            pltpu.VMEM((_C,), jnp.int32),
            pltpu.VMEM((_C,), jnp.int32),
            pltpu.VMEM((_C, 2 * _E), jnp.float32),
            pltpu.VMEM((_C, 2 * _E), jnp.float32),
            pltpu.VMEM((_C, _E), jnp.float32),
            pltpu.VMEM((_C, _E), jnp.float32),
            pltpu.SemaphoreType.DMA,
            pltpu.SemaphoreType.DMA,
            pltpu.SemaphoreType.DMA,
            pltpu.SemaphoreType.DMA,
        ],
    )(_emb_body)
    return f(xf, tok2, pos2)


def kernel(x, token_emb, pos_emb):
    xf = x.reshape(-1)
    tok2 = token_emb.reshape(token_emb.shape[0] // 2, 2 * _E)
    pos2 = jnp.concatenate([pos_emb, pos_emb], axis=0)
    out = _emb(xf, tok2, pos2)
    return out.reshape(x.shape[0], x.shape[1], _E)


# speedup vs baseline: 1.4454x; 1.4026x over previous
import functools

import jax
import jax.numpy as jnp
from jax import lax
from jax.experimental import pallas as pl
from jax.experimental.pallas import tpu as pltpu
from jax.experimental.pallas import tpu_sc as plsc

_B = 4096
_S = 200
_E = 64
_V = 1000000
_N = _B * _S
_NW = 32
_PER_W = _N // _NW
_C = _S
_NCHUNK = _PER_W // _C

def _emb_body(x_hbm, tpad_hbm, posT_hbm, out_hbm,
              posT_v, pos_v, idx0, idx1, rows0, rows1, outb,
              gsem0, gsem1, osem, isem0, isem1):
    idx = (idx0, idx1)
    rows = (rows0, rows1)
    gsem = (gsem0, gsem1)
    isem = (isem0, isem1)

    wid = lax.axis_index("s") * 2 + lax.axis_index("c")
    base = wid * _PER_W

    iot = lax.iota(jnp.int32, 16)
    pltpu.sync_copy(posT_hbm.at[:, pl.ds(0, 128)], posT_v)

    def prow_a(e, c2):
        col = jnp.full((16,), e, jnp.int32)
        for lg in range(8):
            plsc.store_scatter(pos_v, [iot + lg * 16, col],
                               posT_v[e, pl.ds(lg * 16, 16)])
        return c2

    lax.fori_loop(0, _E, prow_a, 0)
    pltpu.sync_copy(posT_hbm.at[:, pl.ds(128, 128)], posT_v)

    def prow_b(e, c2):
        col = jnp.full((16,), e, jnp.int32)
        for lg in range(5):
            s_idx = iot + (128 + lg * 16)
            v = posT_v[e, pl.ds(lg * 16, 16)]
            if lg < 4:
                plsc.store_scatter(pos_v, [s_idx, col], v)
            else:
                plsc.store_scatter(pos_v, [s_idx, col], v, mask=iot < 8)
        return c2

    lax.fori_loop(0, _E, prow_b, 0)

    pltpu.sync_copy(x_hbm.at[pl.ds(base, _C)], idx0)
    pltpu.async_copy(tpad_hbm.at[idx0], rows0, gsem0)
    pltpu.async_copy(x_hbm.at[pl.ds(base + _C, _C)], idx1, isem1)

    def pair(g2, carry):
        for b in range(2):
            g = g2 * 2 + b
            nb = 1 - b

            pltpu.make_async_copy(tpad_hbm.at[idx[b]], rows[b],
                                  gsem[b]).wait()

            @pl.when(g + 2 < _NCHUNK)
            def _nextidx():
                pltpu.async_copy(x_hbm.at[pl.ds(base + (g + 2) * _C, _C)],
                                 idx[b], isem[b])

            @pl.when(g + 1 < _NCHUNK)
            def _nextgather():
                pltpu.make_async_copy(x_hbm.at[pl.ds(base, _C)], idx[nb],
                                      isem[nb]).wait()
                pltpu.async_copy(tpad_hbm.at[idx[nb]], rows[nb], gsem[nb])

            @pl.when(g >= 1)
            def _drain():
                pltpu.make_async_copy(outb, out_hbm.at[pl.ds(base, _C)],
                                      osem).wait()

            def compact(r, c2):
                for c in range(_E // 16):
                    sl = pl.ds(c * 16, 16)
                    outb[r, sl] = rows[b][r, sl] + pos_v[r, sl]
                return c2

            lax.fori_loop(0, _C, compact, 0)
            pltpu.async_copy(outb, out_hbm.at[pl.ds(base + g * _C, _C)],
                             osem)
        return carry

    lax.fori_loop(0, _NCHUNK // 2, pair, 0)
    pltpu.make_async_copy(outb, out_hbm.at[pl.ds(base, _C)], osem).wait()


@jax.jit
def _run(x, token_emb, pos_emb):
    mesh = plsc.VectorSubcoreMesh(core_axis_name="c", subcore_axis_name="s")
    tpad = jnp.pad(token_emb, ((0, 0), (0, _E)))

    emb = functools.partial(
        pl.kernel,
        mesh=mesh,
        compiler_params=pltpu.CompilerParams(needs_layout_passes=False),
        out_type=jax.ShapeDtypeStruct((_N, _E), jnp.float32),
        scratch_types=[
            pltpu.VMEM((_E, 128), jnp.float32),
            pltpu.VMEM((_S, _E), jnp.float32),
            pltpu.VMEM((_C,), jnp.int32),
            pltpu.VMEM((_C,), jnp.int32),
            pltpu.VMEM((_C, 2 * _E), jnp.float32),
            pltpu.VMEM((_C, 2 * _E), jnp.float32),
            pltpu.VMEM((_C, _E), jnp.float32),
            pltpu.SemaphoreType.DMA,
            pltpu.SemaphoreType.DMA,
            pltpu.SemaphoreType.DMA,
            pltpu.SemaphoreType.DMA,
            pltpu.SemaphoreType.DMA,
        ],
    )(_emb_body)
    posTp = jnp.pad(pos_emb.T, ((0, 0), (0, 256 - _S)))
    out = emb(x.reshape(-1), tpad, posTp)
    return out.reshape(x.shape[0], x.shape[1], _E)


def kernel(x, token_emb, pos_emb):
    return _run(x, token_emb, pos_emb)


# speedup vs baseline: 1.4468x; 1.0010x over previous
import functools

import jax
import jax.numpy as jnp
from jax import lax
from jax.experimental import pallas as pl
from jax.experimental.pallas import tpu as pltpu
from jax.experimental.pallas import tpu_sc as plsc

_B = 4096
_S = 200
_E = 64
_V = 1000000
_N = _B * _S
_NW = 32
_PER_W = _N // _NW
_C = _S
_NCHUNK = _PER_W // _C

def _emb_body(x_hbm, tpad_hbm, posT_hbm, out_hbm,
              posT_v, pos_v, idx0, idx1, rows0, rows1, outb,
              gsem0, gsem1, osem, isem0, isem1):
    idx = (idx0, idx1)
    rows = (rows0, rows1)
    gsem = (gsem0, gsem1)
    isem = (isem0, isem1)

    wid = lax.axis_index("s") * 2 + lax.axis_index("c")
    base = wid * _PER_W

    iot = lax.iota(jnp.int32, 16)
    pltpu.sync_copy(posT_hbm.at[:, pl.ds(0, 128)], posT_v)

    def prow_a(e, c2):
        col = jnp.full((16,), e, jnp.int32)
        for lg in range(8):
            plsc.store_scatter(pos_v, [iot + lg * 16, col],
                               posT_v[e, pl.ds(lg * 16, 16)])
        return c2

    lax.fori_loop(0, _E, prow_a, 0)
    pltpu.sync_copy(posT_hbm.at[:, pl.ds(128, 128)], posT_v)

    def prow_b(e, c2):
        col = jnp.full((16,), e, jnp.int32)
        for lg in range(5):
            s_idx = iot + (128 + lg * 16)
            v = posT_v[e, pl.ds(lg * 16, 16)]
            if lg < 4:
                plsc.store_scatter(pos_v, [s_idx, col], v)
            else:
                plsc.store_scatter(pos_v, [s_idx, col], v, mask=iot < 8)
        return c2

    lax.fori_loop(0, _E, prow_b, 0)

    pltpu.sync_copy(x_hbm.at[pl.ds(base, _C)], idx0)
    pltpu.async_copy(tpad_hbm.at[idx0], rows0, gsem0)
    pltpu.async_copy(x_hbm.at[pl.ds(base + _C, _C)], idx1, isem1)

    def pair(g2, carry):
        for b in range(2):
            g = g2 * 2 + b
            nb = 1 - b

            pltpu.make_async_copy(tpad_hbm.at[idx[b]], rows[b],
                                  gsem[b]).wait()

            @pl.when(g + 2 < _NCHUNK)
            def _nextidx():
                pltpu.async_copy(x_hbm.at[pl.ds(base + (g + 2) * _C, _C)],
                                 idx[b], isem[b])

            @pl.when(g + 1 < _NCHUNK)
            def _nextgather():
                pltpu.make_async_copy(x_hbm.at[pl.ds(base, _C)], idx[nb],
                                      isem[nb]).wait()
                pltpu.async_copy(tpad_hbm.at[idx[nb]], rows[nb], gsem[nb])

            @pl.when(g >= 1)
            def _drain():
                pltpu.make_async_copy(outb, out_hbm.at[pl.ds(base, _C)],
                                      osem).wait()

            def compact(r4, c2):
                for u in range(4):
                    r = r4 * 4 + u
                    for c in range(_E // 16):
                        sl = pl.ds(c * 16, 16)
                        outb[r, sl] = rows[b][r, sl] + pos_v[r, sl]
                return c2

            lax.fori_loop(0, _C // 4, compact, 0)
            pltpu.async_copy(outb, out_hbm.at[pl.ds(base + g * _C, _C)],
                             osem)
        return carry

    lax.fori_loop(0, _NCHUNK // 2, pair, 0)
    pltpu.make_async_copy(outb, out_hbm.at[pl.ds(base, _C)], osem).wait()


@jax.jit
def _run(x, token_emb, pos_emb):
    mesh = plsc.VectorSubcoreMesh(core_axis_name="c", subcore_axis_name="s")
    tpad = jnp.pad(token_emb, ((0, 0), (0, _E)))

    emb = functools.partial(
        pl.kernel,
        mesh=mesh,
        compiler_params=pltpu.CompilerParams(needs_layout_passes=False),
        out_type=jax.ShapeDtypeStruct((_N, _E), jnp.float32),
        scratch_types=[
            pltpu.VMEM((_E, 128), jnp.float32),
            pltpu.VMEM((_S, _E), jnp.float32),
            pltpu.VMEM((_C,), jnp.int32),
            pltpu.VMEM((_C,), jnp.int32),
            pltpu.VMEM((_C, 2 * _E), jnp.float32),
            pltpu.VMEM((_C, 2 * _E), jnp.float32),
            pltpu.VMEM((_C, _E), jnp.float32),
            pltpu.SemaphoreType.DMA,
            pltpu.SemaphoreType.DMA,
            pltpu.SemaphoreType.DMA,
            pltpu.SemaphoreType.DMA,
            pltpu.SemaphoreType.DMA,
        ],
    )(_emb_body)
    posTp = jnp.pad(pos_emb.T, ((0, 0), (0, 256 - _S)))
    out = emb(x.reshape(-1), tpad, posTp)
    return out.reshape(x.shape[0], x.shape[1], _E)


def kernel(x, token_emb, pos_emb):
    return _run(x, token_emb, pos_emb)
